# baseline (speedup 1.0000x reference)
import functools

import jax
import jax.numpy as jnp
from jax import lax
from jax.experimental import pallas as pl
from jax.experimental.pallas import tpu as pltpu
from jax.experimental.pallas import tpu_sc as plsc

L = 256
H = 512
EPS = 1e-5
NC, NS = 2, 16
NW = NC * NS
CW = 128

_sc_mesh = functools.partial(
    plsc.VectorSubcoreMesh, core_axis_name="c", subcore_axis_name="s")


def _linear(x, w, b, block_rows, out_dtype=jnp.float32):
    r, din = x.shape
    dout = w.shape[1]
    nb = r // block_rows

    def body(x_ref, w_ref, b_ref, o_ref):
        o_ref[...] = (jnp.dot(x_ref[...], w_ref[...],
                              preferred_element_type=jnp.float32)
                      + b_ref[...]).astype(out_dtype)

    return pl.pallas_call(
        body,
        grid=(nb,),
        in_specs=[
            pl.BlockSpec((block_rows, din), lambda i: (i, 0)),
            pl.BlockSpec((din, dout), lambda i: (0, 0)),
            pl.BlockSpec((1, dout), lambda i: (0, 0)),
        ],
        out_specs=pl.BlockSpec((block_rows, dout), lambda i: (i, 0)),
        out_shape=jax.ShapeDtypeStruct((r, dout), out_dtype),
    )(x, w, b.reshape(1, dout))


def _node_layer1(plain, seg, w1_parts, b1, e_total, block_rows):
    r = plain[0].shape[0] if plain else seg[0][0].shape[0]
    dh = b1.shape[0]
    n_plain, n_seg = len(plain), len(seg)
    nb = r // block_rows

    def body(*refs):
        i = 0
        acc = None
        for k in range(n_plain):
            a = refs[i][...]
            i += 1
            part = jnp.dot(a, refs[i][...], preferred_element_type=jnp.float32)
            i += 1
            acc = part if acc is None else acc + part
        for k in range(n_seg):
            s = refs[i][...]
            cnt = refs[i + 1][:, 0:1]
            stats = refs[i + 2][...]
            i += 3
            mean = stats[0:1, :] / e_total
            var = (stats[1:2, :] - e_total * mean * mean) / (e_total - 1)
            inv = lax.rsqrt(var + EPS)
            m = jnp.where(cnt > 0.0,
                          (s / jnp.maximum(cnt, 1.0) - mean) * inv, 0.0)
            part = jnp.dot(m, refs[i][...], preferred_element_type=jnp.float32)
            i += 1
            acc = part if acc is None else acc + part
        b1_ref, h_ref = refs[i:i + 2]
        h_ref[...] = jnp.maximum(acc + b1_ref[...], 0.0)

    args, specs = [], []
    for k in range(n_plain):
        d = plain[k].shape[1]
        args += [plain[k], w1_parts[k]]
        specs += [pl.BlockSpec((block_rows, d), lambda i: (i, 0)),
                  pl.BlockSpec((d, dh), lambda i: (0, 0))]
    for k in range(n_seg):
        d = seg[k][0].shape[1]
        args += [seg[k][0], seg[k][1], seg[k][2], w1_parts[n_plain + k]]
        specs += [pl.BlockSpec((block_rows, d), lambda i: (i, 0)),
                  pl.BlockSpec((block_rows, 16), lambda i: (i, 0)),
                  pl.BlockSpec((2, L), lambda i: (0, 0)),
                  pl.BlockSpec((d, dh), lambda i: (0, 0))]
    args += [b1.reshape(1, dh)]
    specs += [pl.BlockSpec((1, dh), lambda i: (0, 0))]

    return pl.pallas_call(
        body,
        grid=(nb,),
        in_specs=specs,
        out_specs=pl.BlockSpec((block_rows, dh), lambda i: (i, 0)),
        out_shape=jax.ShapeDtypeStruct((r, dh), jnp.float32),
    )(*args)


def _node_layer2(h, w2, b2):
    r, dh = h.shape

    def body(h_ref, w2_ref, b2_ref, o_ref):
        h = h_ref[...]
        hm = jnp.mean(h, axis=0, keepdims=True)
        hd = h - hm
        hv = jnp.sum(hd * hd, axis=0, keepdims=True) / (r - 1)
        hn = hd * lax.rsqrt(hv + EPS)
        o = jnp.maximum(jnp.dot(hn, w2_ref[...],
                                preferred_element_type=jnp.float32)
                        + b2_ref[...], 0.0)
        om = jnp.mean(o, axis=0, keepdims=True)
        od = o - om
        ov = jnp.sum(od * od, axis=0, keepdims=True) / (r - 1)
        o_ref[...] = od * lax.rsqrt(ov + EPS)

    return pl.pallas_call(
        body,
        out_shape=jax.ShapeDtypeStruct((r, L), jnp.float32),
    )(h, w2, b2.reshape(1, L))


def _node_mlp(plain, seg, w1_parts, b1, w2, b2, e_total, block_rows=2000):
    h = _node_layer1(plain, seg, w1_parts, b1, e_total, block_rows)
    return _node_layer2(h, w2, b2)


def _edge_mlp(g, ea, w1b, b1, w2, b2, block_rows):
    e, dh = g.shape
    dea = ea.shape[1]
    nb = e // block_rows

    def body(g_ref, ea_ref, w1b_ref, b1_ref, w2_ref, b2_ref,
             o_ref, stats_ref, s_sum, s_ssq, s_mean, s_inv, o_sum, o_ssq):
        p = pl.program_id(0)
        b = pl.program_id(1)
        h = jnp.maximum(
            g_ref[...].astype(jnp.float32)
            + jnp.dot(ea_ref[...], w1b_ref[...],
                      preferred_element_type=jnp.float32)
            + b1_ref[...], 0.0)

        @pl.when(p == 0)
        def _():
            sh = jnp.sum(h, axis=0, keepdims=True)
            ssh = jnp.sum(h * h, axis=0, keepdims=True)
            s_sum[...] = jnp.where(b == 0, sh, s_sum[...] + sh)
            s_ssq[...] = jnp.where(b == 0, ssh, s_ssq[...] + ssh)

        @pl.when((p == 1) & (b == 0))
        def _():
            mean = s_sum[...] / e
            var = (s_ssq[...] - e * mean * mean) / (e - 1)
            s_mean[...] = mean
            s_inv[...] = lax.rsqrt(var + EPS)

        @pl.when(p == 1)
        def _():
            hn = (h - s_mean[...]) * s_inv[...]
            o = jnp.maximum(jnp.dot(hn, w2_ref[...],
                                    preferred_element_type=jnp.float32)
                            + b2_ref[...], 0.0)
            o_ref[...] = o
            so = jnp.sum(o, axis=0, keepdims=True)
            sso = jnp.sum(o * o, axis=0, keepdims=True)
            o_sum[...] = jnp.where(b == 0, so, o_sum[...] + so)
            o_ssq[...] = jnp.where(b == 0, sso, o_ssq[...] + sso)

            @pl.when(b == nb - 1)
            def _():
                stats_ref[0:1, :] = o_sum[...]
                stats_ref[1:2, :] = o_ssq[...]

    return pl.pallas_call(
        body,
        grid=(2, nb),
        in_specs=[
            pl.BlockSpec((block_rows, dh), lambda p, b: (b, 0)),
            pl.BlockSpec((block_rows, dea), lambda p, b: (b, 0)),
            pl.BlockSpec((dea, dh), lambda p, b: (0, 0)),
            pl.BlockSpec((1, dh), lambda p, b: (0, 0)),
            pl.BlockSpec((dh, L), lambda p, b: (0, 0)),
            pl.BlockSpec((1, L), lambda p, b: (0, 0)),
        ],
        out_specs=[
            pl.BlockSpec((block_rows, L), lambda p, b: (b, 0)),
            pl.BlockSpec((2, L), lambda p, b: (0, 0)),
        ],
        out_shape=[
            jax.ShapeDtypeStruct((e, L), jnp.float32),
            jax.ShapeDtypeStruct((2, L), jnp.float32),
        ],
        scratch_shapes=[pltpu.VMEM((1, dh), jnp.float32)] * 4
        + [pltpu.VMEM((1, L), jnp.float32)] * 2,
    )(g, ea, w1b, b1.reshape(1, dh), w2, b2.reshape(1, L))


def _sc_gather(table, idx):
    e = idx.shape[0]
    d = table.shape[1]
    per_w = e // NW
    ch = 80
    n_ch = per_w // ch

    n_pairs = (n_ch - 1) // 2
    dt = table.dtype

    @functools.partial(
        pl.kernel,
        out_type=jax.ShapeDtypeStruct((e, d), dt),
        mesh=_sc_mesh(),
        scratch_types=[
            pltpu.VMEM((per_w,), jnp.int32),
            pltpu.VMEM((ch, d), dt),
            pltpu.VMEM((ch, d), dt),
            pltpu.SemaphoreType.DMA,
            pltpu.SemaphoreType.DMA,
        ],
    )
    def k(table_hbm, idx_hbm, out_hbm, idx_v, buf_a, buf_b, sem_a, sem_b):
        wid = lax.axis_index("s") * NC + lax.axis_index("c")
        base = wid * per_w
        pltpu.sync_copy(idx_hbm.at[pl.ds(base, per_w)], idx_v)

        def gath(c, buf, sem):
            return pltpu.make_async_copy(
                table_hbm.at[idx_v.at[pl.ds(c * ch, ch)]], buf, sem)

        def wout(c, buf):
            pltpu.sync_copy(buf, out_hbm.at[pl.ds(base + c * ch, ch)])

        gath(0, buf_a, sem_a).start()

        def body(k2, carry):
            c0 = 2 * k2
            gath(c0 + 1, buf_b, sem_b).start()
            gath(c0, buf_a, sem_a).wait()
            wout(c0, buf_a)
            gath(c0 + 2, buf_a, sem_a).start()
            gath(c0 + 1, buf_b, sem_b).wait()
            wout(c0 + 1, buf_b)
            return carry

        lax.fori_loop(0, n_pairs, body, 0)
        c0 = 2 * n_pairs
        if n_ch - c0 == 2:
            gath(c0 + 1, buf_b, sem_b).start()
            gath(c0, buf_a, sem_a).wait()
            wout(c0, buf_a)
            gath(c0 + 1, buf_b, sem_b).wait()
            wout(c0 + 1, buf_b)
        else:
            gath(c0, buf_a, sem_a).wait()
            wout(c0, buf_a)

    return k(table, idx)


def _sc_scatter(o, idx, n_seg_p):
    e = idx.shape[0]
    per_t = e // NS
    ch = 80
    n_ch = per_t // ch
    stripe = n_seg_p // NS
    idx3 = idx.reshape(NS * n_ch, ch)
    zeros = jnp.zeros((n_seg_p, CW), jnp.float32)

    n_pairs = (n_ch - 1) // 2

    @functools.partial(
        pl.kernel,
        out_type=jax.ShapeDtypeStruct((n_seg_p, 2 * CW), jnp.float32),
        mesh=_sc_mesh(),
        scratch_types=[
            pltpu.VMEM((ch,), jnp.int32),
            pltpu.VMEM((ch,), jnp.int32),
            pltpu.VMEM((ch, CW), jnp.float32),
            pltpu.VMEM((ch, CW), jnp.float32),
            pltpu.SemaphoreType.DMA,
            pltpu.SemaphoreType.DMA,
            pltpu.SemaphoreType.DMA,
            pltpu.SemaphoreType.DMA,
            pltpu.MemorySpace.VMEM_SHARED((n_seg_p, CW), jnp.float32),
        ],
    )
    def k(o_hbm, idx_hbm, zeros_hbm, s_out, idx_a, idx_b, buf_a, buf_b,
          sem_a, sem_b, sem_ia, sem_ib, acc):
        cid = lax.axis_index("c")
        sid = lax.axis_index("s")
        pltpu.sync_copy(zeros_hbm.at[pl.ds(sid * stripe, stripe)],
                        acc.at[pl.ds(sid * stripe, stripe)])
        plsc.subcore_barrier()

        e0 = sid * per_t
        r0 = sid * n_ch

        def load(c, buf, sem):
            return pltpu.make_async_copy(
                o_hbm.at[pl.ds(e0 + c * ch, ch), pl.ds(cid * CW, CW)],
                buf, sem)

        def ldidx(c, ib, sem):
            return pltpu.make_async_copy(idx_hbm.at[r0 + c], ib, sem)

        def add(ib, buf):
            pltpu.sync_copy(buf, acc.at[ib], add=True)

        load(0, buf_a, sem_a).start()
        ldidx(0, idx_a, sem_ia).start()

        def body(k2, carry):
            c0 = 2 * k2
            load(c0 + 1, buf_b, sem_b).start()
            ldidx(c0 + 1, idx_b, sem_ib).start()
            load(c0, buf_a, sem_a).wait()
            ldidx(c0, idx_a, sem_ia).wait()
            add(idx_a, buf_a)
            load(c0 + 2, buf_a, sem_a).start()
            ldidx(c0 + 2, idx_a, sem_ia).start()
            load(c0 + 1, buf_b, sem_b).wait()
            ldidx(c0 + 1, idx_b, sem_ib).wait()
            add(idx_b, buf_b)
            return carry

        lax.fori_loop(0, n_pairs, body, 0)
        c0 = 2 * n_pairs
        if n_ch - c0 == 2:
            load(c0 + 1, buf_b, sem_b).start()
            ldidx(c0 + 1, idx_b, sem_ib).start()
            load(c0, buf_a, sem_a).wait()
            ldidx(c0, idx_a, sem_ia).wait()
            add(idx_a, buf_a)
            load(c0 + 1, buf_b, sem_b).wait()
            ldidx(c0 + 1, idx_b, sem_ib).wait()
            add(idx_b, buf_b)
        else:
            load(c0, buf_a, sem_a).wait()
            ldidx(c0, idx_a, sem_ia).wait()
            add(idx_a, buf_a)
        plsc.subcore_barrier()
        pltpu.sync_copy(acc.at[pl.ds(sid * stripe, stripe)],
                        s_out.at[pl.ds(sid * stripe, stripe),
                                 pl.ds(cid * CW, CW)])

    return k(o, idx3, zeros)


def _sc_count(idx, n_seg_p):
    e = idx.shape[0]
    per_t = e // NS
    ch = 80
    n_ch = per_t // ch
    stripe = n_seg_p // NS
    n_pairs = (n_ch - 1) // 2
    idx3 = idx.reshape(NS * n_ch, ch)
    zeros = jnp.zeros((n_seg_p, CW), jnp.float32)
    ones = jnp.ones((ch, CW), jnp.float32)

    @functools.partial(
        pl.kernel,
        out_type=jax.ShapeDtypeStruct((n_seg_p, 2 * CW), jnp.float32),
        mesh=_sc_mesh(),
        scratch_types=[
            pltpu.VMEM((ch,), jnp.int32),
            pltpu.VMEM((ch,), jnp.int32),
            pltpu.VMEM((ch, CW), jnp.float32),
            pltpu.SemaphoreType.DMA,
            pltpu.SemaphoreType.DMA,
            pltpu.MemorySpace.VMEM_SHARED((n_seg_p, CW), jnp.float32),
        ],
    )
    def k(idx_hbm, zeros_hbm, ones_hbm, s_out, idx_a, idx_b, buf,
          sem_ia, sem_ib, acc):
        cid = lax.axis_index("c")
        sid = lax.axis_index("s")
        pltpu.sync_copy(zeros_hbm.at[pl.ds(sid * stripe, stripe)],
                        acc.at[pl.ds(sid * stripe, stripe)])
        pltpu.sync_copy(ones_hbm, buf)
        plsc.subcore_barrier()

        r0 = sid * n_ch

        def ldidx(c, ib, sem):
            return pltpu.make_async_copy(idx_hbm.at[r0 + c], ib, sem)

        def add(ib):
            pltpu.sync_copy(buf, acc.at[ib], add=True)

        ldidx(0, idx_a, sem_ia).start()

        def body(k2, carry):
            c0 = 2 * k2
            ldidx(c0 + 1, idx_b, sem_ib).start()
            ldidx(c0, idx_a, sem_ia).wait()
            add(idx_a)
            ldidx(c0 + 2, idx_a, sem_ia).start()
            ldidx(c0 + 1, idx_b, sem_ib).wait()
            add(idx_b)
            return carry

        lax.fori_loop(0, n_pairs, body, 0)
        c0 = 2 * n_pairs
        if n_ch - c0 == 2:
            ldidx(c0 + 1, idx_b, sem_ib).start()
            ldidx(c0, idx_a, sem_ia).wait()
            add(idx_a)
            ldidx(c0 + 1, idx_b, sem_ib).wait()
            add(idx_b)
        else:
            ldidx(c0, idx_a, sem_ia).wait()
            add(idx_a)
        plsc.subcore_barrier()
        pltpu.sync_copy(acc.at[pl.ds(sid * stripe, stripe)],
                        s_out.at[pl.ds(sid * stripe, stripe),
                                 pl.ds(cid * CW, CW)])

    return k(idx3, zeros, ones)


def kernel(node_ins, edge_index_ins, edge_attr_ins, node_label,
           edge_index_cross, edge_attr_cross, params):
    p = params
    n = node_ins.shape[0]
    nl = node_label.shape[0]
    e = edge_index_ins.shape[1]
    row = edge_index_ins[0]
    col = edge_index_ins[1]
    row_c = edge_index_cross[0]
    col_c = edge_index_cross[1]
    n_p = ((n + 8 * NS - 1) // (8 * NS)) * (8 * NS)
    nl_p = ((nl + 8 * NS - 1) // (8 * NS)) * (8 * NS)

    cnt_col = _sc_count(col, n_p)[:n, :16]
    cnt_rowc = _sc_count(row_c, n_p)[:n, :16]
    cnt_colc = _sc_count(col_c, nl_p)[:nl, :16]

    x1 = _node_mlp([node_ins], [], [p['enc_W1']], p['enc_b1'],
                   p['enc_W2'], p['enc_b2'], e)

    def as_i32(t):
        return jax.lax.bitcast_convert_type(
            t.reshape(t.shape[0], t.shape[1] // 2, 2), jnp.int32)

    def as_bf16(g):
        return jax.lax.bitcast_convert_type(g, jnp.bfloat16).reshape(
            g.shape[0], g.shape[1] * 2)

    t1 = _linear(x1, p['mlp1_W1'][:L], p['mlp1_b1'], 2000, jnp.bfloat16)
    g1 = as_bf16(_sc_gather(as_i32(t1), row))
    o1, st1 = _edge_mlp(g1, edge_attr_ins, p['mlp1_W1'][L:], p['mlp1_b1'],
                        p['mlp1_W2'], p['mlp1_b2'], 2000)
    s1 = _sc_scatter(o1, col, n_p)[:n]
    x2 = _node_mlp([x1], [(s1, cnt_col, st1)],
                   [p['mlp2_W1'][:L], p['mlp2_W1'][L:]], p['mlp2_b1'],
                   p['mlp2_W2'], p['mlp2_b2'], e)

    t2 = _linear(x2, p['inner_W1'][:L], p['inner_b1'], 2000, jnp.bfloat16)
    g2 = as_bf16(_sc_gather(as_i32(t2), row))
    o2, st2 = _edge_mlp(g2, edge_attr_ins, p['inner_W1'][L:], p['inner_b1'],
                        p['inner_W2'], p['inner_b2'], 2000)
    s2 = _sc_scatter(o2, col, n_p)[:n]

    t3 = _linear(node_label, p['inter_W1'][:128], p['inter_b1'], 1000, jnp.bfloat16)
    g3 = as_bf16(_sc_gather(as_i32(t3), col_c))
    o3, st3 = _edge_mlp(g3, edge_attr_cross, p['inter_W1'][128:],
                        p['inter_b1'], p['inter_W2'], p['inter_b2'], 2000)
    s3 = _sc_scatter(o3, row_c, n_p)[:n]

    x3 = _node_mlp([x2], [(s2, cnt_col, st2), (s3, cnt_rowc, st3)],
                   [p['ins_W1'][:L], p['ins_W1'][L:2 * L], p['ins_W1'][2 * L:]],
                   p['ins_b1'], p['ins_W2'], p['ins_b2'], e)

    t4 = _linear(x3, p['label_inter_W1'][:L], p['label_inter_b1'], 2000, jnp.bfloat16)
    g4 = as_bf16(_sc_gather(as_i32(t4), row_c))
    o4, st4 = _edge_mlp(g4, edge_attr_cross, p['label_inter_W1'][L:],
                        p['label_inter_b1'], p['label_inter_W2'],
                        p['label_inter_b2'], 2000)
    s4 = _sc_scatter(o4, col_c, nl_p)[:nl]

    y = _node_mlp([node_label], [(s4, cnt_colc, st4)],
                  [p['label_W1'][:128], p['label_W1'][128:]], p['label_b1'],
                  p['label_W2'], p['label_b2'], e, block_rows=1000)

    return (x3, y)


# speedup vs baseline: 2.5807x; 2.5807x over previous
import functools

import jax
import jax.numpy as jnp
from jax import lax
from jax.experimental import pallas as pl
from jax.experimental.pallas import tpu as pltpu
from jax.experimental.pallas import tpu_sc as plsc

L = 256
H = 512
EPS = 1e-5
NC, NS = 2, 16
NW = NC * NS
CW = 128

_sc_mesh = functools.partial(
    plsc.VectorSubcoreMesh, core_axis_name="c", subcore_axis_name="s")


def _linear(x, w, b, block_rows, out_dtype=jnp.float32):
    r, din = x.shape
    dout = w.shape[1]
    nb = r // block_rows

    def body(x_ref, w_ref, b_ref, o_ref):
        o_ref[...] = (jnp.dot(x_ref[...], w_ref[...],
                              preferred_element_type=jnp.float32)
                      + b_ref[...]).astype(out_dtype)

    return pl.pallas_call(
        body,
        grid=(nb,),
        in_specs=[
            pl.BlockSpec((block_rows, din), lambda i: (i, 0)),
            pl.BlockSpec((din, dout), lambda i: (0, 0)),
            pl.BlockSpec((1, dout), lambda i: (0, 0)),
        ],
        out_specs=pl.BlockSpec((block_rows, dout), lambda i: (i, 0)),
        out_shape=jax.ShapeDtypeStruct((r, dout), out_dtype),
    )(x, w, b.reshape(1, dout))


def _node_layer1(plain, seg, w1_parts, b1, e_total, block_rows):
    r = plain[0].shape[0] if plain else seg[0][0].shape[0]
    dh = b1.shape[0]
    n_plain, n_seg = len(plain), len(seg)
    nb = r // block_rows

    def body(*refs):
        i = 0
        acc = None
        for k in range(n_plain):
            a = refs[i][...]
            i += 1
            part = jnp.dot(a, refs[i][...], preferred_element_type=jnp.float32)
            i += 1
            acc = part if acc is None else acc + part
        for k in range(n_seg):
            s = refs[i][...]
            cnt = refs[i + 1][:, 0:1]
            stats = refs[i + 2][...]
            i += 3
            mean = stats[0:1, :] / e_total
            var = (stats[1:2, :] - e_total * mean * mean) / (e_total - 1)
            inv = lax.rsqrt(var + EPS)
            m = jnp.where(cnt > 0.0,
                          (s / jnp.maximum(cnt, 1.0) - mean) * inv, 0.0)
            part = jnp.dot(m, refs[i][...], preferred_element_type=jnp.float32)
            i += 1
            acc = part if acc is None else acc + part
        b1_ref, h_ref = refs[i:i + 2]
        h_ref[...] = jnp.maximum(acc + b1_ref[...], 0.0)

    args, specs = [], []
    for k in range(n_plain):
        d = plain[k].shape[1]
        args += [plain[k], w1_parts[k]]
        specs += [pl.BlockSpec((block_rows, d), lambda i: (i, 0)),
                  pl.BlockSpec((d, dh), lambda i: (0, 0))]
    for k in range(n_seg):
        d = seg[k][0].shape[1]
        args += [seg[k][0], seg[k][1], seg[k][2], w1_parts[n_plain + k]]
        specs += [pl.BlockSpec((block_rows, d), lambda i: (i, 0)),
                  pl.BlockSpec((block_rows, 16), lambda i: (i, 0)),
                  pl.BlockSpec((2, L), lambda i: (0, 0)),
                  pl.BlockSpec((d, dh), lambda i: (0, 0))]
    args += [b1.reshape(1, dh)]
    specs += [pl.BlockSpec((1, dh), lambda i: (0, 0))]

    return pl.pallas_call(
        body,
        grid=(nb,),
        in_specs=specs,
        out_specs=pl.BlockSpec((block_rows, dh), lambda i: (i, 0)),
        out_shape=jax.ShapeDtypeStruct((r, dh), jnp.float32),
    )(*args)


def _node_layer2(h, w2, b2):
    r, dh = h.shape

    def body(h_ref, w2_ref, b2_ref, o_ref):
        h = h_ref[...]
        hm = jnp.mean(h, axis=0, keepdims=True)
        hd = h - hm
        hv = jnp.sum(hd * hd, axis=0, keepdims=True) / (r - 1)
        hn = hd * lax.rsqrt(hv + EPS)
        o = jnp.maximum(jnp.dot(hn, w2_ref[...],
                                preferred_element_type=jnp.float32)
                        + b2_ref[...], 0.0)
        om = jnp.mean(o, axis=0, keepdims=True)
        od = o - om
        ov = jnp.sum(od * od, axis=0, keepdims=True) / (r - 1)
        o_ref[...] = od * lax.rsqrt(ov + EPS)

    return pl.pallas_call(
        body,
        out_shape=jax.ShapeDtypeStruct((r, L), jnp.float32),
    )(h, w2, b2.reshape(1, L))


def _node_mlp(plain, seg, w1_parts, b1, w2, b2, e_total, block_rows=2000):
    h = _node_layer1(plain, seg, w1_parts, b1, e_total, block_rows)
    return _node_layer2(h, w2, b2)


def _edge_mlp(g, ea, w1b, b1, w2, b2, block_rows):
    e, dh = g.shape
    dea = ea.shape[1]
    nb = e // block_rows

    def body(g_ref, ea_ref, w1b_ref, b1_ref, w2_ref, b2_ref,
             o_ref, stats_ref, s_sum, s_ssq, s_mean, s_inv, o_sum, o_ssq):
        p = pl.program_id(0)
        b = pl.program_id(1)
        h = jnp.maximum(
            g_ref[...].astype(jnp.float32)
            + jnp.dot(ea_ref[...], w1b_ref[...],
                      preferred_element_type=jnp.float32)
            + b1_ref[...], 0.0)

        @pl.when(p == 0)
        def _():
            sh = jnp.sum(h, axis=0, keepdims=True)
            ssh = jnp.sum(h * h, axis=0, keepdims=True)
            s_sum[...] = jnp.where(b == 0, sh, s_sum[...] + sh)
            s_ssq[...] = jnp.where(b == 0, ssh, s_ssq[...] + ssh)

        @pl.when((p == 1) & (b == 0))
        def _():
            mean = s_sum[...] / e
            var = (s_ssq[...] - e * mean * mean) / (e - 1)
            s_mean[...] = mean
            s_inv[...] = lax.rsqrt(var + EPS)

        @pl.when(p == 1)
        def _():
            hn = (h - s_mean[...]) * s_inv[...]
            o = jnp.maximum(jnp.dot(hn, w2_ref[...],
                                    preferred_element_type=jnp.float32)
                            + b2_ref[...], 0.0)
            o_ref[...] = o
            so = jnp.sum(o, axis=0, keepdims=True)
            sso = jnp.sum(o * o, axis=0, keepdims=True)
            o_sum[...] = jnp.where(b == 0, so, o_sum[...] + so)
            o_ssq[...] = jnp.where(b == 0, sso, o_ssq[...] + sso)

            @pl.when(b == nb - 1)
            def _():
                stats_ref[0:1, :] = o_sum[...]
                stats_ref[1:2, :] = o_ssq[...]

    return pl.pallas_call(
        body,
        grid=(2, nb),
        in_specs=[
            pl.BlockSpec((block_rows, dh), lambda p, b: (b, 0)),
            pl.BlockSpec((block_rows, dea), lambda p, b: (b, 0)),
            pl.BlockSpec((dea, dh), lambda p, b: (0, 0)),
            pl.BlockSpec((1, dh), lambda p, b: (0, 0)),
            pl.BlockSpec((dh, L), lambda p, b: (0, 0)),
            pl.BlockSpec((1, L), lambda p, b: (0, 0)),
        ],
        out_specs=[
            pl.BlockSpec((block_rows, L), lambda p, b: (b, 0)),
            pl.BlockSpec((2, L), lambda p, b: (0, 0)),
        ],
        out_shape=[
            jax.ShapeDtypeStruct((e, L), jnp.float32),
            jax.ShapeDtypeStruct((2, L), jnp.float32),
        ],
        scratch_shapes=[pltpu.VMEM((1, dh), jnp.float32)] * 4
        + [pltpu.VMEM((1, L), jnp.float32)] * 2,
    )(g, ea, w1b, b1.reshape(1, dh), w2, b2.reshape(1, L))


def _sc_gather(table, idx):
    e = idx.shape[0]
    d = table.shape[1]
    per_w = e // NW
    ch = 80
    n_ch = per_w // ch

    n_pairs = (n_ch - 1) // 2
    dt = table.dtype

    @functools.partial(
        pl.kernel,
        out_type=jax.ShapeDtypeStruct((e, d), dt),
        mesh=_sc_mesh(),
        scratch_types=[
            pltpu.VMEM((per_w,), jnp.int32),
            pltpu.VMEM((ch, d), dt),
            pltpu.VMEM((ch, d), dt),
            pltpu.SemaphoreType.DMA,
            pltpu.SemaphoreType.DMA,
        ],
    )
    def k(table_hbm, idx_hbm, out_hbm, idx_v, buf_a, buf_b, sem_a, sem_b):
        wid = lax.axis_index("s") * NC + lax.axis_index("c")
        base = wid * per_w
        pltpu.sync_copy(idx_hbm.at[pl.ds(base, per_w)], idx_v)

        def gath(c, buf, sem):
            return pltpu.make_async_copy(
                table_hbm.at[idx_v.at[pl.ds(c * ch, ch)]], buf, sem)

        def wout(c, buf):
            pltpu.sync_copy(buf, out_hbm.at[pl.ds(base + c * ch, ch)])

        gath(0, buf_a, sem_a).start()

        def body(k2, carry):
            c0 = 2 * k2
            gath(c0 + 1, buf_b, sem_b).start()
            gath(c0, buf_a, sem_a).wait()
            wout(c0, buf_a)
            gath(c0 + 2, buf_a, sem_a).start()
            gath(c0 + 1, buf_b, sem_b).wait()
            wout(c0 + 1, buf_b)
            return carry

        lax.fori_loop(0, n_pairs, body, 0)
        c0 = 2 * n_pairs
        if n_ch - c0 == 2:
            gath(c0 + 1, buf_b, sem_b).start()
            gath(c0, buf_a, sem_a).wait()
            wout(c0, buf_a)
            gath(c0 + 1, buf_b, sem_b).wait()
            wout(c0 + 1, buf_b)
        else:
            gath(c0, buf_a, sem_a).wait()
            wout(c0, buf_a)

    return k(table, idx)


def _sc_scatter(o, idx, n_seg_p):
    e = idx.shape[0]
    per_t = e // NS
    ch = 80
    n_ch = per_t // ch
    stripe = n_seg_p // NS
    idx3 = idx.reshape(NS * n_ch, ch)
    zeros = jnp.zeros((n_seg_p, CW), jnp.float32)

    n_pairs = (n_ch - 1) // 2

    @functools.partial(
        pl.kernel,
        out_type=jax.ShapeDtypeStruct((n_seg_p, 2 * CW), jnp.float32),
        mesh=_sc_mesh(),
        scratch_types=[
            pltpu.VMEM((ch,), jnp.int32),
            pltpu.VMEM((ch,), jnp.int32),
            pltpu.VMEM((ch, CW), jnp.float32),
            pltpu.VMEM((ch, CW), jnp.float32),
            pltpu.SemaphoreType.DMA,
            pltpu.SemaphoreType.DMA,
            pltpu.SemaphoreType.DMA,
            pltpu.SemaphoreType.DMA,
            pltpu.MemorySpace.VMEM_SHARED((n_seg_p, CW), jnp.float32),
        ],
    )
    def k(o_hbm, idx_hbm, zeros_hbm, s_out, idx_a, idx_b, buf_a, buf_b,
          sem_a, sem_b, sem_ia, sem_ib, acc):
        cid = lax.axis_index("c")
        sid = lax.axis_index("s")
        pltpu.sync_copy(zeros_hbm.at[pl.ds(sid * stripe, stripe)],
                        acc.at[pl.ds(sid * stripe, stripe)])
        plsc.subcore_barrier()

        e0 = sid * per_t
        r0 = sid * n_ch

        def load(c, buf, sem):
            return pltpu.make_async_copy(
                o_hbm.at[pl.ds(e0 + c * ch, ch), pl.ds(cid * CW, CW)],
                buf, sem)

        def ldidx(c, ib, sem):
            return pltpu.make_async_copy(idx_hbm.at[r0 + c], ib, sem)

        def add(ib, buf):
            pltpu.sync_copy(buf, acc.at[ib], add=True)

        load(0, buf_a, sem_a).start()
        ldidx(0, idx_a, sem_ia).start()

        def body(k2, carry):
            c0 = 2 * k2
            load(c0 + 1, buf_b, sem_b).start()
            ldidx(c0 + 1, idx_b, sem_ib).start()
            load(c0, buf_a, sem_a).wait()
            ldidx(c0, idx_a, sem_ia).wait()
            add(idx_a, buf_a)
            load(c0 + 2, buf_a, sem_a).start()
            ldidx(c0 + 2, idx_a, sem_ia).start()
            load(c0 + 1, buf_b, sem_b).wait()
            ldidx(c0 + 1, idx_b, sem_ib).wait()
            add(idx_b, buf_b)
            return carry

        lax.fori_loop(0, n_pairs, body, 0)
        c0 = 2 * n_pairs
        if n_ch - c0 == 2:
            load(c0 + 1, buf_b, sem_b).start()
            ldidx(c0 + 1, idx_b, sem_ib).start()
            load(c0, buf_a, sem_a).wait()
            ldidx(c0, idx_a, sem_ia).wait()
            add(idx_a, buf_a)
            load(c0 + 1, buf_b, sem_b).wait()
            ldidx(c0 + 1, idx_b, sem_ib).wait()
            add(idx_b, buf_b)
        else:
            load(c0, buf_a, sem_a).wait()
            ldidx(c0, idx_a, sem_ia).wait()
            add(idx_a, buf_a)
        plsc.subcore_barrier()
        pltpu.sync_copy(acc.at[pl.ds(sid * stripe, stripe)],
                        s_out.at[pl.ds(sid * stripe, stripe),
                                 pl.ds(cid * CW, CW)])

    return k(o, idx3, zeros)


def _sc_count(idx, n_seg_p):
    e = idx.shape[0]
    per_t = e // NS
    ch = 80
    n_ch = per_t // ch
    stripe = n_seg_p // NS
    n_pairs = (n_ch - 1) // 2
    idx3 = idx.reshape(NS * n_ch, ch)
    zeros = jnp.zeros((n_seg_p, CW), jnp.float32)
    ones = jnp.ones((ch, CW), jnp.float32)

    @functools.partial(
        pl.kernel,
        out_type=jax.ShapeDtypeStruct((n_seg_p, 2 * CW), jnp.float32),
        mesh=_sc_mesh(),
        scratch_types=[
            pltpu.VMEM((ch,), jnp.int32),
            pltpu.VMEM((ch,), jnp.int32),
            pltpu.VMEM((ch, CW), jnp.float32),
            pltpu.SemaphoreType.DMA,
            pltpu.SemaphoreType.DMA,
            pltpu.MemorySpace.VMEM_SHARED((n_seg_p, CW), jnp.float32),
        ],
    )
    def k(idx_hbm, zeros_hbm, ones_hbm, s_out, idx_a, idx_b, buf,
          sem_ia, sem_ib, acc):
        cid = lax.axis_index("c")
        sid = lax.axis_index("s")
        pltpu.sync_copy(zeros_hbm.at[pl.ds(sid * stripe, stripe)],
                        acc.at[pl.ds(sid * stripe, stripe)])
        pltpu.sync_copy(ones_hbm, buf)
        plsc.subcore_barrier()

        r0 = sid * n_ch

        def ldidx(c, ib, sem):
            return pltpu.make_async_copy(idx_hbm.at[r0 + c], ib, sem)

        def add(ib):
            pltpu.sync_copy(buf, acc.at[ib], add=True)

        ldidx(0, idx_a, sem_ia).start()

        def body(k2, carry):
            c0 = 2 * k2
            ldidx(c0 + 1, idx_b, sem_ib).start()
            ldidx(c0, idx_a, sem_ia).wait()
            add(idx_a)
            ldidx(c0 + 2, idx_a, sem_ia).start()
            ldidx(c0 + 1, idx_b, sem_ib).wait()
            add(idx_b)
            return carry

        lax.fori_loop(0, n_pairs, body, 0)
        c0 = 2 * n_pairs
        if n_ch - c0 == 2:
            ldidx(c0 + 1, idx_b, sem_ib).start()
            ldidx(c0, idx_a, sem_ia).wait()
            add(idx_a)
            ldidx(c0 + 1, idx_b, sem_ib).wait()
            add(idx_b)
        else:
            ldidx(c0, idx_a, sem_ia).wait()
            add(idx_a)
        plsc.subcore_barrier()
        pltpu.sync_copy(acc.at[pl.ds(sid * stripe, stripe)],
                        s_out.at[pl.ds(sid * stripe, stripe),
                                 pl.ds(cid * CW, CW)])

    return k(idx3, zeros, ones)


def kernel(node_ins, edge_index_ins, edge_attr_ins, node_label,
           edge_index_cross, edge_attr_cross, params):
    p = params
    n = node_ins.shape[0]
    nl = node_label.shape[0]
    e = edge_index_ins.shape[1]
    row = edge_index_ins[0]
    col = edge_index_ins[1]
    row_c = edge_index_cross[0]
    col_c = edge_index_cross[1]
    n_p = ((n + 8 * NS - 1) // (8 * NS)) * (8 * NS)
    nl_p = ((nl + 8 * NS - 1) // (8 * NS)) * (8 * NS)

    cnt_col = _sc_count(col, n_p)[:n, :16]
    cnt_rowc = _sc_count(row_c, n_p)[:n, :16]
    cnt_colc = _sc_count(col_c, nl_p)[:nl, :16]

    x1 = _node_mlp([node_ins], [], [p['enc_W1']], p['enc_b1'],
                   p['enc_W2'], p['enc_b2'], e)

    t1 = _linear(x1, p['mlp1_W1'][:L], p['mlp1_b1'], 2000)
    g1 = _sc_gather(t1, row)
    o1, st1 = _edge_mlp(g1, edge_attr_ins, p['mlp1_W1'][L:], p['mlp1_b1'],
                        p['mlp1_W2'], p['mlp1_b2'], 2000)
    s1 = _sc_scatter(o1, col, n_p)[:n]
    x2 = _node_mlp([x1], [(s1, cnt_col, st1)],
                   [p['mlp2_W1'][:L], p['mlp2_W1'][L:]], p['mlp2_b1'],
                   p['mlp2_W2'], p['mlp2_b2'], e)

    t2 = _linear(x2, p['inner_W1'][:L], p['inner_b1'], 2000)
    g2 = _sc_gather(t2, row)
    o2, st2 = _edge_mlp(g2, edge_attr_ins, p['inner_W1'][L:], p['inner_b1'],
                        p['inner_W2'], p['inner_b2'], 2000)
    s2 = _sc_scatter(o2, col, n_p)[:n]

    t3 = _linear(node_label, p['inter_W1'][:128], p['inter_b1'], 1000)
    g3 = _sc_gather(t3, col_c)
    o3, st3 = _edge_mlp(g3, edge_attr_cross, p['inter_W1'][128:],
                        p['inter_b1'], p['inter_W2'], p['inter_b2'], 2000)
    s3 = _sc_scatter(o3, row_c, n_p)[:n]

    x3 = _node_mlp([x2], [(s2, cnt_col, st2), (s3, cnt_rowc, st3)],
                   [p['ins_W1'][:L], p['ins_W1'][L:2 * L], p['ins_W1'][2 * L:]],
                   p['ins_b1'], p['ins_W2'], p['ins_b2'], e)

    t4 = _linear(x3, p['label_inter_W1'][:L], p['label_inter_b1'], 2000)
    g4 = _sc_gather(t4, row_c)
    o4, st4 = _edge_mlp(g4, edge_attr_cross, p['label_inter_W1'][L:],
                        p['label_inter_b1'], p['label_inter_W2'],
                        p['label_inter_b2'], 2000)
    s4 = _sc_scatter(o4, col_c, nl_p)[:nl]

    y = _node_mlp([node_label], [(s4, cnt_colc, st4)],
                  [p['label_W1'][:128], p['label_W1'][128:]], p['label_b1'],
                  p['label_W2'], p['label_b2'], e, block_rows=1000)

    return (x3, y)
